# Initial kernel scaffold; baseline (speedup 1.0000x reference)
import jax
import jax.numpy as jnp
from jax.experimental import pallas as pl


def kernel(X, adj, W, b, S):
    raise NotImplementedError("write your pallas kernel here")


# speedup vs baseline: 2.8034x; 2.8034x over previous
import functools

import jax
import jax.numpy as jnp
from jax.experimental import pallas as pl
from jax.experimental.pallas import tpu as pltpu

_NEG_INF = -3.0e38


def _prologue_body(x_ref, w_ref, b_ref, s2w_ref, xp_ref, s2_ref, cmax_ref):
    t = pl.program_id(0)
    xp = jax.lax.dot_general(
        x_ref[...], w_ref[...],
        dimension_numbers=(((1,), (1,)), ((), ())),
        preferred_element_type=jnp.float32,
    ) + b_ref[...]
    xp_ref[...] = xp
    s2 = jnp.sum(xp * s2w_ref[...], axis=1, keepdims=True)
    s2_ref[...] = s2

    @pl.when(t == 0)
    def _():
        cmax_ref[...] = jnp.full((1, 1), _NEG_INF, jnp.float32)

    cmax_ref[...] = jnp.maximum(cmax_ref[...],
                                jnp.max(s2, axis=(0, 1), keepdims=True))


def _vbuild_body(xp_ref, s2_ref, cmax_ref, vc_ref):
    e = jnp.exp(s2_ref[...] - cmax_ref[...])
    v = xp_ref[...] * e
    t, f = v.shape
    vc_ref[...] = jnp.concatenate(
        [v, e, jnp.zeros((t, f - 1), jnp.float32)], axis=1)


def _main_body(adj_ref, vc_ref, vself_ref, out_ref, *, out_f):
    res = jnp.dot(adj_ref[...], vc_ref[...],
                  preferred_element_type=jnp.float32)
    num = res[:, :out_f] + vself_ref[:, :out_f]
    den = res[:, out_f:out_f + 1] + vself_ref[:, out_f:out_f + 1]
    out_ref[...] = jax.nn.sigmoid(num / den)


def kernel(X, adj, W, b, S):
    n, in_f = X.shape
    out_f = W.shape[0]

    tp = 1000
    ti = 400

    s2w = S[out_f:].reshape(1, out_f)
    b2 = b.reshape(1, out_f)

    xp, s2, cmax = pl.pallas_call(
        _prologue_body,
        grid=(n // tp,),
        in_specs=[
            pl.BlockSpec((tp, in_f), lambda t: (t, 0)),
            pl.BlockSpec((out_f, in_f), lambda t: (0, 0)),
            pl.BlockSpec((1, out_f), lambda t: (0, 0)),
            pl.BlockSpec((1, out_f), lambda t: (0, 0)),
        ],
        out_specs=[
            pl.BlockSpec((tp, out_f), lambda t: (t, 0)),
            pl.BlockSpec((tp, 1), lambda t: (t, 0)),
            pl.BlockSpec((1, 1), lambda t: (0, 0)),
        ],
        out_shape=[
            jax.ShapeDtypeStruct((n, out_f), jnp.float32),
            jax.ShapeDtypeStruct((n, 1), jnp.float32),
            jax.ShapeDtypeStruct((1, 1), jnp.float32),
        ],
    )(X, W, b2, s2w)

    vc = pl.pallas_call(
        _vbuild_body,
        grid=(n // tp,),
        in_specs=[
            pl.BlockSpec((tp, out_f), lambda t: (t, 0)),
            pl.BlockSpec((tp, 1), lambda t: (t, 0)),
            pl.BlockSpec((1, 1), lambda t: (0, 0)),
        ],
        out_specs=pl.BlockSpec((tp, 2 * out_f), lambda t: (t, 0)),
        out_shape=jax.ShapeDtypeStruct((n, 2 * out_f), jnp.float32),
    )(xp, s2, cmax)

    out = pl.pallas_call(
        functools.partial(_main_body, out_f=out_f),
        grid=(n // ti,),
        in_specs=[
            pl.BlockSpec((ti, n), lambda i: (i, 0)),
            pl.BlockSpec((n, 2 * out_f), lambda i: (0, 0)),
            pl.BlockSpec((ti, 2 * out_f), lambda i: (i, 0)),
        ],
        out_specs=pl.BlockSpec((ti, out_f), lambda i: (i, 0)),
        out_shape=jax.ShapeDtypeStruct((n, out_f), jnp.float32),
    )(adj, vc, vc)

    return out
